# baseline (speedup 1.0000x reference)
import functools
import math

import jax
import jax.numpy as jnp
from jax import lax
from jax.experimental import pallas as pl
from jax.experimental.pallas import tpu as pltpu
from jax.experimental.pallas import tpu_sc as plsc

_SCALE = 64.0
_MARGIN = 0.5
_COS_M = math.cos(_MARGIN)
_SIN_M = math.sin(_MARGIN)
_THETA = math.cos(math.pi - _MARGIN)
_SINMM = math.sin(math.pi - _MARGIN) * _MARGIN

_B = 1024
_V = 100000
_L = 16

_BSC = 512
_BTC = _B - _BSC

_NC = 2
_NS = 16
_NW = _NC * _NS
_RPW = _BSC // _NW
_RG = 8
_NG = _RPW // _RG

_CC = 6400
_NCH = 16
_CLAST = 93568
_CT = 32
_T0 = _V - _CT
_NVEC = _CC // _L


def _margin16(t):
    x = 1.0 - t * t
    i = lax.bitcast_convert_type(x, jnp.int32)
    i = jnp.int32(0x5F3759DF) - lax.shift_right_logical(i, 1)
    r = lax.bitcast_convert_type(i, jnp.float32)
    for _ in range(3):
        r = r * (1.5 - (0.5 * x) * r * r)
    sin_t = x * r
    ctm = t * _COS_M - sin_t * _SIN_M
    return jnp.where(t > _THETA, ctm, t - _SINMM)


def _sc_body(lg_hbm, lab_hbm, out_hbm, lab_v, buf0, buf1, tbuf,
             isem0, isem1, osem0, osem1):
    wid = lax.axis_index("s") * _NC + lax.axis_index("c")
    base = wid * _RPW
    pltpu.sync_copy(lab_hbm.at[pl.ds(base, _RPW)], lab_v)
    labs = []
    for h in range(_RPW // _L):
        lv = lab_v[pl.ds(h * _L, _L)]
        labs.extend(lv[k] for k in range(_L))

    def in_cp(r0, c0, buf, sem_):
        return pltpu.make_async_copy(
            lg_hbm.at[pl.ds(r0, _RG), pl.ds(c0, _CC)], buf, sem_)

    def out_cp(r0, c0, buf, sem_):
        return pltpu.make_async_copy(
            buf, out_hbm.at[pl.ds(r0, _RG), pl.ds(c0, _CC)], sem_)

    def fix_and_scale(g, c0, buf, clen=_CC):
        for rr in range(_RG):
            lab = labs[g * _RG + rr]

            @pl.when((lab >= c0) & (lab < c0 + clen))
            def _():
                off = lab - c0
                a = (off // _L) * _L
                v16 = buf[rr, pl.ds(a, _L)]
                m16 = _margin16(v16)
                idx16 = lax.iota(jnp.int32, _L) + a
                buf[rr, pl.ds(a, _L)] = jnp.where(idx16 == off, m16, v16)

        for rr in range(_RG):
            @plsc.parallel_loop(0, clen // _L, unroll=8)
            def vec_body(j):
                buf[rr, pl.ds(j * _L, _L)] = buf[rr, pl.ds(j * _L, _L)] * _SCALE

    def col0(c):
        return jnp.minimum(c * _CC, jnp.int32(_CLAST))

    for g in range(_NG):
        r0 = base + g * _RG
        in_cp(r0, 0, buf0, isem0).start()

        @pl.loop(0, _NCH // 2)
        def pipe_body(cc):
            c0 = col0(cc * 2)
            c1 = col0(cc * 2 + 1)

            @pl.when(cc > 0)
            def _():
                out_cp(r0, col0(cc * 2 - 1), buf1, osem1).wait()

            in_cp(r0, c1, buf1, isem1).start()
            in_cp(r0, c0, buf0, isem0).wait()
            fix_and_scale(g, c0, buf0)
            out_cp(r0, c0, buf0, osem0).start()
            in_cp(r0, c1, buf1, isem1).wait()
            fix_and_scale(g, c1, buf1)
            out_cp(r0, c1, buf1, osem1).start()

            @pl.when(cc + 1 < _NCH // 2)
            def _():
                out_cp(r0, c0, buf0, osem0).wait()
                in_cp(r0, col0(cc * 2 + 2), buf0, isem0).start()

        out_cp(r0, col0(_NCH - 2), buf0, osem0).wait()
        out_cp(r0, col0(_NCH - 1), buf1, osem1).wait()

        pltpu.sync_copy(lg_hbm.at[pl.ds(r0, _RG), pl.ds(_T0, _CT)], tbuf)
        fix_and_scale(g, jnp.int32(_T0), tbuf, clen=_CT)
        pltpu.sync_copy(tbuf, out_hbm.at[pl.ds(r0, _RG), pl.ds(_T0, _CT)])


_sc_run = functools.partial(
    pl.kernel,
    mesh=plsc.VectorSubcoreMesh(core_axis_name="c", subcore_axis_name="s"),
    out_type=jax.ShapeDtypeStruct((_BSC, _V), jnp.float32),
    scratch_types=[
        pltpu.VMEM((_RPW,), jnp.int32),
        pltpu.VMEM((_RG, _CC), jnp.float32),
        pltpu.VMEM((_RG, _CC), jnp.float32),
        pltpu.VMEM((_RG, _CT), jnp.float32),
        pltpu.SemaphoreType.DMA,
        pltpu.SemaphoreType.DMA,
        pltpu.SemaphoreType.DMA,
        pltpu.SemaphoreType.DMA,
    ],
)


_TBB = 16


def _tc_body(lab_ref, logit_ref, out_ref):
    x = logit_ref[...]
    sin_t = jnp.sqrt(1.0 - x * x)
    ctm = x * _COS_M - sin_t * _SIN_M
    fx = jnp.where(x > _THETA, ctm, x - _SINMM)
    cols = lax.broadcasted_iota(jnp.int32, x.shape, 1)
    mask = cols == lab_ref[...]
    out_ref[...] = jnp.where(mask, fx, x) * _SCALE


def kernel(logits, labels):
    sc_out = _sc_run(_sc_body)(logits, labels)
    off = _BSC // _TBB
    tc_out = pl.pallas_call(
        _tc_body,
        grid=(_BTC // _TBB,),
        in_specs=[
            pl.BlockSpec((_TBB, 1), lambda i: (i + off, 0)),
            pl.BlockSpec((_TBB, _V), lambda i: (i + off, 0)),
        ],
        out_specs=pl.BlockSpec((_TBB, _V), lambda i: (i, 0)),
        out_shape=jax.ShapeDtypeStruct((_BTC, _V), jnp.float32),
    )(labels.reshape(_B, 1), logits)
    return jnp.concatenate([sc_out, tc_out], axis=0)


# speedup vs baseline: 1.1495x; 1.1495x over previous
import functools
import math

import jax
import jax.numpy as jnp
from jax import lax
from jax.experimental import pallas as pl
from jax.experimental.pallas import tpu as pltpu
from jax.experimental.pallas import tpu_sc as plsc

_SCALE = 64.0
_MARGIN = 0.5
_COS_M = math.cos(_MARGIN)
_SIN_M = math.sin(_MARGIN)
_THETA = math.cos(math.pi - _MARGIN)
_SINMM = math.sin(math.pi - _MARGIN) * _MARGIN

_B = 1024
_V = 100000
_L = 16

_NC = 2
_NS = 16
_NW = _NC * _NS
_RPW = _B // _NW
_RG = 8
_NG = _RPW // _RG

_CC = 6400
_NCH = 16
_CLAST = 93568
_CT = 32
_T0 = _V - _CT
_NVEC = _CC // _L


def _margin16(t):
    x = 1.0 - t * t
    i = lax.bitcast_convert_type(x, jnp.int32)
    i = jnp.int32(0x5F3759DF) - lax.shift_right_logical(i, 1)
    r = lax.bitcast_convert_type(i, jnp.float32)
    for _ in range(3):
        r = r * (1.5 - (0.5 * x) * r * r)
    sin_t = x * r
    ctm = t * _COS_M - sin_t * _SIN_M
    return jnp.where(t > _THETA, ctm, t - _SINMM)


def _sc_body(lg_hbm, lab_hbm, out_hbm, lab_v, buf0, buf1, tbuf,
             isem0, isem1, osem0, osem1):
    wid = lax.axis_index("s") * _NC + lax.axis_index("c")
    base = wid * _RPW
    pltpu.sync_copy(lab_hbm.at[pl.ds(base, _RPW)], lab_v)
    labs = []
    for h in range(_RPW // _L):
        lv = lab_v[pl.ds(h * _L, _L)]
        labs.extend(lv[k] for k in range(_L))

    def in_cp(r0, c0, buf, sem_):
        return pltpu.make_async_copy(
            lg_hbm.at[pl.ds(r0, _RG), pl.ds(c0, _CC)], buf, sem_)

    def out_cp(r0, c0, buf, sem_):
        return pltpu.make_async_copy(
            buf, out_hbm.at[pl.ds(r0, _RG), pl.ds(c0, _CC)], sem_)

    def fix_and_scale(g, c0, buf, clen=_CC):
        for rr in range(_RG):
            lab = labs[g * _RG + rr]

            @pl.when((lab >= c0) & (lab < c0 + clen))
            def _():
                off = lab - c0
                a = (off // _L) * _L
                v16 = buf[rr, pl.ds(a, _L)]
                m16 = _margin16(v16)
                idx16 = lax.iota(jnp.int32, _L) + a
                buf[rr, pl.ds(a, _L)] = jnp.where(idx16 == off, m16, v16)

        for rr in range(_RG):
            @plsc.parallel_loop(0, clen // _L, unroll=8)
            def vec_body(j):
                buf[rr, pl.ds(j * _L, _L)] = buf[rr, pl.ds(j * _L, _L)] * _SCALE

    def col0(c):
        return jnp.minimum(c * _CC, jnp.int32(_CLAST))

    for g in range(_NG):
        r0 = base + g * _RG
        in_cp(r0, 0, buf0, isem0).start()

        @pl.loop(0, _NCH // 2)
        def pipe_body(cc):
            c0 = col0(cc * 2)
            c1 = col0(cc * 2 + 1)

            @pl.when(cc > 0)
            def _():
                out_cp(r0, col0(cc * 2 - 1), buf1, osem1).wait()

            in_cp(r0, c1, buf1, isem1).start()
            in_cp(r0, c0, buf0, isem0).wait()
            fix_and_scale(g, c0, buf0)
            out_cp(r0, c0, buf0, osem0).start()
            in_cp(r0, c1, buf1, isem1).wait()
            fix_and_scale(g, c1, buf1)
            out_cp(r0, c1, buf1, osem1).start()

            @pl.when(cc + 1 < _NCH // 2)
            def _():
                out_cp(r0, c0, buf0, osem0).wait()
                in_cp(r0, col0(cc * 2 + 2), buf0, isem0).start()

        out_cp(r0, col0(_NCH - 2), buf0, osem0).wait()
        out_cp(r0, col0(_NCH - 1), buf1, osem1).wait()

        pltpu.sync_copy(lg_hbm.at[pl.ds(r0, _RG), pl.ds(_T0, _CT)], tbuf)
        fix_and_scale(g, jnp.int32(_T0), tbuf, clen=_CT)
        pltpu.sync_copy(tbuf, out_hbm.at[pl.ds(r0, _RG), pl.ds(_T0, _CT)])


_sc_run = functools.partial(
    pl.kernel,
    mesh=plsc.VectorSubcoreMesh(core_axis_name="c", subcore_axis_name="s"),
    out_type=jax.ShapeDtypeStruct((_B, _V), jnp.float32),
    scratch_types=[
        pltpu.VMEM((_RPW,), jnp.int32),
        pltpu.VMEM((_RG, _CC), jnp.float32),
        pltpu.VMEM((_RG, _CC), jnp.float32),
        pltpu.VMEM((_RG, _CT), jnp.float32),
        pltpu.SemaphoreType.DMA,
        pltpu.SemaphoreType.DMA,
        pltpu.SemaphoreType.DMA,
        pltpu.SemaphoreType.DMA,
    ],
)


def kernel(logits, labels):
    return _sc_run(_sc_body)(logits, labels)
